# baseline (speedup 1.0000x reference)
import functools

import jax
import jax.numpy as jnp
from jax import lax
from jax.experimental import pallas as pl
from jax.experimental.pallas import tpu as pltpu
from jax.experimental.pallas import tpu_sc as plsc

_B, _L, _V, _H, _D = 128, 2048, 64, 64, 3
_HW = 80
_NC, _NS, _NL = 1, 16, 16
_NW = _NC * _NS
_RPW = _B // _NW


@functools.lru_cache(maxsize=None)
def _hist_kernel():
    mesh = plsc.VectorSubcoreMesh(core_axis_name="c", subcore_axis_name="s",
                                  num_cores=1)

    _BS = _V + 1

    @functools.partial(
        pl.kernel,
        mesh=mesh,
        compiler_params=pltpu.CompilerParams(needs_layout_passes=False),
        out_type=jax.ShapeDtypeStruct((_B, _HW), jnp.int32),
        scratch_types=[
            pltpu.VMEM((_RPW, _L), jnp.int32),
            pltpu.VMEM((_NL, _BS), jnp.int32),
            pltpu.VMEM((_RPW, _HW), jnp.int32),
        ],
    )
    def hist(seq_hbm, hist_hbm, seq_v, bins_v, hist_v):
        wid = lax.axis_index("s") * _NC + lax.axis_index("c")
        base = wid * _RPW
        pltpu.sync_copy(seq_hbm.at[pl.ds(base, _RPW)], seq_v)
        lanes = lax.iota(jnp.int32, _NL)
        ones = jnp.ones((_NL,), jnp.int32)
        zeros = jnp.zeros((_NL,), jnp.int32)
        unroll = 8
        for i in range(_RPW):
            for r in range(_NL):
                for c in range(_V // _NL):
                    bins_v[r, pl.ds(c * _NL, _NL)] = zeros

            def body(c, carry):
                off = c * (_NL * unroll)
                vals = [seq_v[i, pl.ds(off + u * _NL, _NL)]
                        for u in range(unroll)]
                for v in vals:
                    plsc.addupdate_scatter(bins_v, [lanes, v], ones)
                return carry

            lax.fori_loop(0, _L // (_NL * unroll), body, 0)
            for c in range(_V // _NL):
                acc = bins_v[0, pl.ds(c * _NL, _NL)]
                for r in range(1, _NL):
                    acc = acc + bins_v[r, pl.ds(c * _NL, _NL)]
                hist_v[i, pl.ds(c * _NL, _NL)] = acc
            hist_v[i, pl.ds(_V, _NL)] = seq_v[i, pl.ds(_L - _NL, _NL)]
        pltpu.sync_copy(hist_v, hist_hbm.at[pl.ds(base, _RPW)])

    return hist


def _tc_body(hist_ref, embed_ref, w1_ref, b1_ref, w2_ref, b2_ref,
             lng_ref, lnb_ref, gw_ref, dw_ref, db_ref, qw_ref, qb_ref,
             ow_ref, ob_ref, logits_ref, dom_ref):
    f32 = jnp.float32
    b1 = b1_ref[...].reshape(1, -1)
    b2 = b2_ref[...].reshape(1, -1)
    lng = lng_ref[...].reshape(1, -1)
    lnb = lnb_ref[...].reshape(1, -1)
    db = db_ref[...].reshape(1, -1)
    qb = qb_ref[...].reshape(1, -1)
    ob = ob_ref[...].reshape(1, -1)
    emb = embed_ref[...]
    t = jnp.dot(emb, w1_ref[...], preferred_element_type=f32) + b1
    t = jnp.maximum(t, 0.0)
    t = jnp.dot(t, w2_ref[...], preferred_element_type=f32) + b2
    x = emb + t
    mu = jnp.mean(x, axis=1, keepdims=True)
    xc = x - mu
    var = jnp.mean(xc * xc, axis=1, keepdims=True)
    h_tab = xc * lax.rsqrt(var + 1e-5) * lng + lnb

    histw = hist_ref[...]
    hist = histw[:, :_V]
    last = histw[:, _HW - 1:_HW]
    histf = hist.astype(f32)
    present = hist > 0
    vio = lax.broadcasted_iota(jnp.int32, (_B, _V), 1)
    ohl = (last == vio).astype(f32)
    h_last = jnp.dot(ohl, h_tab, preferred_element_type=f32)
    dom_logits = jnp.dot(h_last, dw_ref[...], preferred_element_type=f32) + db
    dom_ref[...] = dom_logits
    dmx = jnp.max(dom_logits, axis=1, keepdims=True)
    de = jnp.exp(dom_logits - dmx)
    dprobs = de / jnp.sum(de, axis=1, keepdims=True)
    q = jnp.dot(h_last, qw_ref[...], preferred_element_type=f32) + qb

    gvals = lax.dot_general(gw_ref[...], h_tab, (((1,), (1,)), ((), ())),
                            preferred_element_type=f32)
    neg = f32(-1e30)
    slots, scores = [], []
    for d in range(_D):
        gv = gvals[d:d + 1, :]
        masked = jnp.where(present, gv, neg)
        m1 = jnp.max(masked, axis=1, keepdims=True)
        oh1 = jnp.where((masked == m1) & present, 1.0, 0.0)
        slot1 = jnp.dot(oh1, h_tab, preferred_element_type=f32)
        cnt1 = jnp.sum(oh1 * histf, axis=1, keepdims=True)
        masked2 = jnp.where(masked < m1, masked, neg)
        m2 = jnp.max(masked2, axis=1, keepdims=True)
        oh2 = jnp.where((masked2 == m2) & (masked2 > neg * 0.5), 1.0, 0.0)
        slot2b = jnp.dot(oh2, h_tab, preferred_element_type=f32)
        slot2 = jnp.where(cnt1 >= 2.0, slot1, slot2b)
        p = dprobs[:, d:d + 1]
        for s in (slot1 * p, slot2 * p):
            slots.append(s)
            scores.append(jnp.sum(s * q, axis=1, keepdims=True) * f32(0.125))

    smx = scores[0]
    for s in scores[1:]:
        smx = jnp.maximum(smx, s)
    es = [jnp.exp(s - smx) for s in scores]
    z = es[0]
    for e in es[1:]:
        z = z + e
    pooled = (es[0] / z) * slots[0]
    for e, s in zip(es[1:], slots[1:]):
        pooled = pooled + (e / z) * s
    logits_ref[...] = jnp.dot(pooled, ow_ref[...], preferred_element_type=f32) + ob


def kernel(seq, embed, w1, b1, w2, b2, ln_g, ln_b, gate_w, gate_b,
           dom_w, dom_b, q_w, q_b, out_w, out_b):
    del gate_b
    hist = _hist_kernel()(seq)
    logits, dom_logits = pl.pallas_call(
        _tc_body,
        out_shape=(jax.ShapeDtypeStruct((_B, _V), jnp.float32),
                   jax.ShapeDtypeStruct((_B, _D), jnp.float32)),
    )(hist, embed, w1, b1, w2, b2, ln_g, ln_b, gate_w, dom_w,
      dom_b, q_w, q_b, out_w, out_b)
    return logits, dom_logits


# speedup vs baseline: 1.0883x; 1.0883x over previous
import functools

import jax
import jax.numpy as jnp
from jax import lax
from jax.experimental import pallas as pl
from jax.experimental.pallas import tpu as pltpu
from jax.experimental.pallas import tpu_sc as plsc

_B, _L, _V, _H, _D = 128, 2048, 64, 64, 3
_HW = 80
_NC, _NS, _NL = 2, 16, 16
_NW = _NC * _NS
_RPW = _B // _NW


@functools.lru_cache(maxsize=None)
def _hist_kernel():
    mesh = plsc.VectorSubcoreMesh(core_axis_name="c", subcore_axis_name="s")

    _BS = _V + 1

    @functools.partial(
        pl.kernel,
        mesh=mesh,
        compiler_params=pltpu.CompilerParams(needs_layout_passes=False),
        out_type=jax.ShapeDtypeStruct((_B, _HW), jnp.int32),
        scratch_types=[
            pltpu.VMEM((_RPW, _L), jnp.int32),
            pltpu.VMEM((_NL, _BS), jnp.int32),
            pltpu.VMEM((_RPW, _HW), jnp.int32),
        ],
    )
    def hist(seq_hbm, hist_hbm, seq_v, bins_v, hist_v):
        wid = lax.axis_index("s") * _NC + lax.axis_index("c")
        base = wid * _RPW
        pltpu.sync_copy(seq_hbm.at[pl.ds(base, _RPW)], seq_v)
        lanes = lax.iota(jnp.int32, _NL)
        ones = jnp.ones((_NL,), jnp.int32)
        zeros = jnp.zeros((_NL,), jnp.int32)
        unroll = 8
        for i in range(_RPW):
            for r in range(_NL):
                for c in range(_V // _NL):
                    bins_v[r, pl.ds(c * _NL, _NL)] = zeros

            def body(c, carry):
                off = c * (_NL * unroll)
                vals = [seq_v[i, pl.ds(off + u * _NL, _NL)]
                        for u in range(unroll)]
                for v in vals:
                    plsc.addupdate_scatter(bins_v, [lanes, v], ones)
                return carry

            lax.fori_loop(0, _L // (_NL * unroll), body, 0)
            for c in range(_V // _NL):
                acc = bins_v[0, pl.ds(c * _NL, _NL)]
                for r in range(1, _NL):
                    acc = acc + bins_v[r, pl.ds(c * _NL, _NL)]
                hist_v[i, pl.ds(c * _NL, _NL)] = acc
            hist_v[i, pl.ds(_V, _NL)] = seq_v[i, pl.ds(_L - _NL, _NL)]
        pltpu.sync_copy(hist_v, hist_hbm.at[pl.ds(base, _RPW)])

    return hist


def _tc_body(hist_ref, embed_ref, w1_ref, b1_ref, w2_ref, b2_ref,
             lng_ref, lnb_ref, gw_ref, dw_ref, db_ref, qw_ref, qb_ref,
             ow_ref, ob_ref, logits_ref, dom_ref):
    f32 = jnp.float32
    b1 = b1_ref[...].reshape(1, -1)
    b2 = b2_ref[...].reshape(1, -1)
    lng = lng_ref[...].reshape(1, -1)
    lnb = lnb_ref[...].reshape(1, -1)
    db = db_ref[...].reshape(1, -1)
    qb = qb_ref[...].reshape(1, -1)
    ob = ob_ref[...].reshape(1, -1)
    emb = embed_ref[...]
    t = jnp.dot(emb, w1_ref[...], preferred_element_type=f32) + b1
    t = jnp.maximum(t, 0.0)
    t = jnp.dot(t, w2_ref[...], preferred_element_type=f32) + b2
    x = emb + t
    mu = jnp.mean(x, axis=1, keepdims=True)
    xc = x - mu
    var = jnp.mean(xc * xc, axis=1, keepdims=True)
    h_tab = xc * lax.rsqrt(var + 1e-5) * lng + lnb

    histw = hist_ref[...]
    hist = histw[:, :_V]
    last = histw[:, _HW - 1:_HW]
    histf = hist.astype(f32)
    present = hist > 0
    vio = lax.broadcasted_iota(jnp.int32, (_B, _V), 1)
    ohl = (last == vio).astype(f32)
    h_last = jnp.dot(ohl, h_tab, preferred_element_type=f32)
    dom_logits = jnp.dot(h_last, dw_ref[...], preferred_element_type=f32) + db
    dom_ref[...] = dom_logits
    dmx = jnp.max(dom_logits, axis=1, keepdims=True)
    de = jnp.exp(dom_logits - dmx)
    dprobs = de / jnp.sum(de, axis=1, keepdims=True)
    q = jnp.dot(h_last, qw_ref[...], preferred_element_type=f32) + qb

    gvals = lax.dot_general(gw_ref[...], h_tab, (((1,), (1,)), ((), ())),
                            preferred_element_type=f32)
    neg = f32(-1e30)
    slots, scores = [], []
    for d in range(_D):
        gv = gvals[d:d + 1, :]
        masked = jnp.where(present, gv, neg)
        m1 = jnp.max(masked, axis=1, keepdims=True)
        oh1 = jnp.where((masked == m1) & present, 1.0, 0.0)
        slot1 = jnp.dot(oh1, h_tab, preferred_element_type=f32)
        cnt1 = jnp.sum(oh1 * histf, axis=1, keepdims=True)
        masked2 = jnp.where(masked < m1, masked, neg)
        m2 = jnp.max(masked2, axis=1, keepdims=True)
        oh2 = jnp.where((masked2 == m2) & (masked2 > neg * 0.5), 1.0, 0.0)
        slot2b = jnp.dot(oh2, h_tab, preferred_element_type=f32)
        slot2 = jnp.where(cnt1 >= 2.0, slot1, slot2b)
        p = dprobs[:, d:d + 1]
        for s in (slot1 * p, slot2 * p):
            slots.append(s)
            scores.append(jnp.sum(s * q, axis=1, keepdims=True) * f32(0.125))

    smx = scores[0]
    for s in scores[1:]:
        smx = jnp.maximum(smx, s)
    es = [jnp.exp(s - smx) for s in scores]
    z = es[0]
    for e in es[1:]:
        z = z + e
    pooled = (es[0] / z) * slots[0]
    for e, s in zip(es[1:], slots[1:]):
        pooled = pooled + (e / z) * s
    logits_ref[...] = jnp.dot(pooled, ow_ref[...], preferred_element_type=f32) + ob


def kernel(seq, embed, w1, b1, w2, b2, ln_g, ln_b, gate_w, gate_b,
           dom_w, dom_b, q_w, q_b, out_w, out_b):
    del gate_b
    hist = _hist_kernel()(seq)
    logits, dom_logits = pl.pallas_call(
        _tc_body,
        out_shape=(jax.ShapeDtypeStruct((_B, _V), jnp.float32),
                   jax.ShapeDtypeStruct((_B, _D), jnp.float32)),
    )(hist, embed, w1, b1, w2, b2, ln_g, ln_b, gate_w, dom_w,
      dom_b, q_w, q_b, out_w, out_b)
    return logits, dom_logits
